# baseline (speedup 1.0000x reference)
import functools

import jax
import jax.numpy as jnp
from jax import lax
from jax.experimental import pallas as pl
from jax.experimental.pallas import tpu as pltpu
from jax.experimental.pallas import tpu_sc as plsc

B = 4096
L = 50
D = 32
LANES = 16
NW = 32
BPW = B // NW


def _embed_kernel_body(u_idx, i_idx, h_idx, u_tab, i_tab, h_tab,
                       out,
                       uidx_v, iidx_v, hT, urows, irows, acc,
                       sem_idx, sem_ui, sem_h):
    wid = lax.axis_index("s") * 2 + lax.axis_index("c")
    base = wid * BPW

    cp_u = pltpu.async_copy(u_idx.at[pl.ds(base, BPW)], uidx_v, sem_idx)
    cp_i = pltpu.async_copy(i_idx.at[pl.ds(base, BPW)], iidx_v, sem_idx)
    cp_h = pltpu.async_copy(h_idx.at[wid], hT, sem_idx)

    zeros = jnp.zeros((LANES,), jnp.float32)

    def zbody(b, carry):
        acc[b, pl.ds(0, LANES)] = zeros
        acc[b, pl.ds(LANES, LANES)] = zeros
        return carry

    lax.fori_loop(0, BPW, zbody, 0)

    cp_u.wait()
    cp_i.wait()

    g_u = pltpu.async_copy(u_tab.at[uidx_v], urows, sem_ui)
    g_i = pltpu.async_copy(i_tab.at[iidx_v], irows, sem_ui)

    cp_h.wait()

    def fire(p, carry):
        pltpu.async_copy(h_tab.at[hT.at[p]], acc, sem_h, add=True)
        return carry

    lax.fori_loop(0, L, fire, 0)

    g_u.wait()
    g_i.wait()
    st_u = pltpu.async_copy(urows, out.at[pl.ds(base, BPW), 0], sem_ui)
    st_i = pltpu.async_copy(irows, out.at[pl.ds(base, BPW), 1], sem_ui)

    def drain(p, carry):
        pltpu.make_async_copy(h_tab.at[hT.at[0]], acc, sem_h).wait()
        return carry

    lax.fori_loop(0, L, drain, 0)

    scale = jnp.full((LANES,), 1.0 / L, jnp.float32)

    def sbody(b, carry):
        acc[b, pl.ds(0, LANES)] = acc[b, pl.ds(0, LANES)] * scale
        acc[b, pl.ds(LANES, LANES)] = acc[b, pl.ds(LANES, LANES)] * scale
        return carry

    lax.fori_loop(0, BPW, sbody, 0)

    pltpu.sync_copy(acc, out.at[pl.ds(base, BPW), 2])
    st_u.wait()
    st_i.wait()


@jax.jit
def kernel(user_idx, item_idx, hist_idx, user_table, item_table, hist_table):
    u_idx = user_idx.reshape(B).astype(jnp.int32)
    i_idx = item_idx.reshape(B).astype(jnp.int32)
    h_idx = hist_idx.astype(jnp.int32).reshape(NW, BPW, L).transpose(0, 2, 1)

    mesh = plsc.VectorSubcoreMesh(core_axis_name="c", subcore_axis_name="s")
    run = functools.partial(
        pl.kernel,
        out_type=jax.ShapeDtypeStruct((B, 3, D), jnp.float32),
        mesh=mesh,
        compiler_params=pltpu.CompilerParams(use_tc_tiling_on_sc=False),
        scratch_types=[
            pltpu.VMEM((BPW,), jnp.int32),
            pltpu.VMEM((BPW,), jnp.int32),
            pltpu.VMEM((L, BPW), jnp.int32),
            pltpu.VMEM((BPW, D), jnp.float32),
            pltpu.VMEM((BPW, D), jnp.float32),
            pltpu.VMEM((BPW, D), jnp.float32),
            pltpu.SemaphoreType.DMA,
            pltpu.SemaphoreType.DMA,
            pltpu.SemaphoreType.DMA,
        ],
    )(_embed_kernel_body)

    return run(u_idx, i_idx, h_idx, user_table, item_table, hist_table)
